# Initial kernel scaffold; baseline (speedup 1.0000x reference)
import jax
import jax.numpy as jnp
from jax.experimental import pallas as pl


def kernel(feats, edge_index, node_graph_ids, W1, b1, gamma1, beta1, W2, b2, gamma2, beta2, Wd, bd, gammad, betad, Wp, bp):
    raise NotImplementedError("write your pallas kernel here")


# speedup vs baseline: 4.2754x; 4.2754x over previous
import functools

import jax
import jax.numpy as jnp
from jax import lax
from jax.experimental import pallas as pl
from jax.experimental.pallas import tpu as pltpu
from jax.experimental.pallas import tpu_sc as plsc

N_NODES = 10000
NR = 10240
N_GRAPHS = 64
MAX_DEG = 10
EPS = 1e-5
E_TOTAL = 320000
NWORK = 32
CHUNK = 128
NCH = 79
TRASH = N_NODES
BN = 1024
GRID = NR // BN
RSUB = NR // 16


def _make_sc_agg(D):
    mesh = plsc.VectorSubcoreMesh(core_axis_name="c", subcore_axis_name="s")
    scratch = [
        pltpu.VMEM((CHUNK,), jnp.int32),
        pltpu.VMEM((CHUNK,), jnp.int32),
        pltpu.VMEM((CHUNK, D), jnp.float32),
        pltpu.VMEM_SHARED((NR, D), jnp.float32),
        pltpu.SemaphoreType.DMA,
    ]

    def body(table, src3, dst3, zrows, out_acc, sidx, didx, rows, acc, sem):
        c = lax.axis_index("c")
        s = lax.axis_index("s")
        w = s * 2 + c
        sub = pl.ds(s * RSUB, RSUB)
        pltpu.sync_copy(zrows.at[sub], acc.at[sub])
        plsc.subcore_barrier()

        def step(j, carry):
            pltpu.sync_copy(src3.at[w, j], sidx)
            pltpu.sync_copy(dst3.at[w, j], didx)
            pltpu.async_copy(table.at[sidx], rows, sem).wait()
            pltpu.sync_copy(rows, acc.at[didx], add=True)
            return carry

        lax.fori_loop(0, NCH, step, 0)
        plsc.subcore_barrier()
        pltpu.sync_copy(acc.at[sub], out_acc.at[c, sub])

    params = (pltpu.CompilerParams(use_tc_tiling_on_sc=False)
              if D != 128 else None)
    return pl.kernel(body, out_type=jax.ShapeDtypeStruct((2, NR, D),
                                                         jnp.float32),
                     mesh=mesh, scratch_types=scratch,
                     compiler_params=params)


def _sc_agg(table, src3, dst3):
    D = table.shape[1]
    k = _make_sc_agg(D)
    return k(table, src3, dst3, jnp.zeros((NR, D), jnp.float32))


def _make_sc_deg():
    mesh = plsc.VectorSubcoreMesh(core_axis_name="c", subcore_axis_name="s")
    scratch = [
        pltpu.VMEM((CHUNK,), jnp.int32),
        pltpu.VMEM((CHUNK, 16), jnp.float32),
        pltpu.VMEM_SHARED((NR, 16), jnp.float32),
        pltpu.VMEM((CHUNK, 16), jnp.float32),
        pltpu.VMEM((CHUNK,), jnp.int32),
        pltpu.VMEM((RSUB // 8, 128), jnp.float32),
        pltpu.SemaphoreType.DMA,
    ]

    def body(dst3, out_deg, didx, ones_v, dacc, zbuf, iv, pbuf, sem):
        c = lax.axis_index("c")
        s = lax.axis_index("s")
        w = s * 2 + c
        vone = jnp.ones((16,), jnp.float32)
        vnil = jnp.zeros((16,), jnp.float32)
        iota = lax.iota(jnp.int32, 16)
        for j in range(CHUNK):
            ones_v[j] = vone
            zbuf[j] = vnil
        for t in range(RSUB // CHUNK):
            for q in range(CHUNK // 16):
                iv[pl.ds(q * 16, 16)] = (s * RSUB + t * CHUNK + q * 16) + iota
            pltpu.sync_copy(zbuf, dacc.at[iv])
        plsc.subcore_barrier()

        def step(j, carry):
            pltpu.sync_copy(dst3.at[w, j], didx)
            pltpu.sync_copy(ones_v, dacc.at[didx], add=True)
            return carry

        lax.fori_loop(0, NCH, step, 0)
        plsc.subcore_barrier()
        for t in range(RSUB // CHUNK):
            for q in range(CHUNK // 16):
                iv[pl.ds(q * 16, 16)] = (s * RSUB + t * CHUNK + q * 16) + iota
            pltpu.async_copy(dacc.at[iv], zbuf, sem).wait()
            for j in range(CHUNK):
                jj = t * CHUNK + j
                pbuf[jj // 8, pl.ds((jj % 8) * 16, 16)] = zbuf[j, pl.ds(0, 16)]
        pltpu.sync_copy(pbuf,
                        out_deg.at[c, pl.ds(s * (RSUB // 8), RSUB // 8)])

    return pl.kernel(body,
                     out_type=jax.ShapeDtypeStruct((2, NR // 8, 128),
                                                   jnp.float32),
                     mesh=mesh, scratch_types=scratch,
                     compiler_params=pltpu.CompilerParams(
                         use_tc_tiling_on_sc=False))


def _sc_deg(dst3):
    return _make_sc_deg()(dst3).reshape(2, NR, 16)


RCH = 8


def _sc_readout(z3, gids_pad):
    D = z3.shape[1]
    nvec = D // 16
    mesh = plsc.VectorSubcoreMesh(core_axis_name="c", subcore_axis_name="s")
    outs = (
        jax.ShapeDtypeStruct((N_GRAPHS, D), jnp.float32),
        jax.ShapeDtypeStruct((N_GRAPHS, D), jnp.float32),
        jax.ShapeDtypeStruct((N_GRAPHS, D), jnp.float32),
        jax.ShapeDtypeStruct((N_GRAPHS, 16), jnp.float32),
    )
    scratch = [
        pltpu.VMEM((NR,), jnp.int32),
        pltpu.VMEM((RCH, D), jnp.float32),
        pltpu.VMEM((D,), jnp.float32),
        pltpu.VMEM((D,), jnp.float32),
        pltpu.VMEM((D,), jnp.float32),
        pltpu.VMEM((16,), jnp.float32),
    ]

    def body(z3h, gidh, osum, omax, omin, ocnt,
             gidv, buf, sacc, xacc, nacc, cntv):
        c = lax.axis_index("c")
        s = lax.axis_index("s")
        w = s * 2 + c
        g0 = w * 2
        pltpu.sync_copy(gidh, gidv)

        one = jnp.ones((16,), jnp.int32)
        nil = jnp.zeros((16,), jnp.int32)

        def cstep(i, carry):
            c0, c1, c2 = carry
            v = gidv[pl.ds(i * 16, 16)]
            c0 = c0 + jnp.where(v < g0, one, nil)
            c1 = c1 + jnp.where(v < (g0 + 1), one, nil)
            c2 = c2 + jnp.where(v < (g0 + 2), one, nil)
            return c0, c1, c2

        c0, c1, c2 = lax.fori_loop(0, NR // 16, cstep, (nil, nil, nil))
        b0 = jnp.int32(0)
        b1 = jnp.int32(0)
        b2 = jnp.int32(0)
        for j in range(16):
            b0 = b0 + c0[j]
            b1 = b1 + c1[j]
            b2 = b2 + c2[j]

        for k in range(2):
            g = g0 + k
            lo = b0 if k == 0 else b1
            hi = b1 if k == 0 else b2
            for t in range(nvec):
                sl = pl.ds(t * 16, 16)
                sacc[sl] = jnp.zeros((16,), jnp.float32)
                xacc[sl] = jnp.full((16,), -3e38, jnp.float32)
                nacc[sl] = jnp.full((16,), 3e38, jnp.float32)

            @pl.loop((lo // RCH) * RCH, hi, step=RCH)
            def rstep(r):
                rr = pl.multiple_of(r, RCH)
                pltpu.sync_copy(z3h.at[pl.ds(rr, RCH)], buf)
                for i in range(RCH):
                    valid = jnp.logical_and(r + i >= lo, r + i < hi)
                    for t in range(nvec):
                        sl = pl.ds(t * 16, 16)
                        row = buf[i, sl]
                        sacc[sl] = sacc[sl] + jnp.where(valid, row, 0.0)
                        xacc[sl] = jnp.maximum(xacc[sl],
                                               jnp.where(valid, row, -3e38))
                        nacc[sl] = jnp.minimum(nacc[sl],
                                               jnp.where(valid, row, 3e38))
            cntv[...] = jnp.full((16,), 1.0, jnp.float32) * (hi - lo).astype(jnp.float32)
            pltpu.sync_copy(sacc, osum.at[g])
            pltpu.sync_copy(xacc, omax.at[g])
            pltpu.sync_copy(nacc, omin.at[g])
            pltpu.sync_copy(cntv, ocnt.at[g])

    k = pl.kernel(body, out_type=outs, mesh=mesh, scratch_types=scratch)
    return k(z3, gids_pad)


def _tc_layer(x, aggp, degp, W, b, stats):
    Din = x.shape[1]
    H = W.shape[2]
    affine = stats is not None

    def body(*refs):
        if affine:
            (x_ref, p0, p1, d0, d1, w_ref, b_ref,
             s_in, sq_in, ga, be, z_ref, s_ref, sq_ref) = refs
        else:
            (x_ref, p0, p1, d0, d1, w_ref, b_ref,
             z_ref, s_ref, sq_ref) = refs
        i = pl.program_id(0)
        msg = x_ref[...] + p0[0] + p1[0]
        deg = d0[0][:, :1] + d1[0][:, :1]
        if affine:
            mean = s_in[...] * (1.0 / N_NODES)
            var = sq_in[...] * (1.0 / N_NODES) - mean * mean
            a = ga[...] * lax.rsqrt(var + EPS)
            cv = be[...] - mean * a
            msg = a * msg + (1.0 + deg) * cv
        didx = jnp.clip(deg, 1.0, float(MAX_DEG)) - 1.0
        acc = jnp.zeros((BN, H), jnp.float32)
        for d in range(MAX_DEG):
            y = jnp.dot(msg, w_ref[d], preferred_element_type=jnp.float32)
            y = y + b_ref[d]
            acc = acc + jnp.where(didx == float(d), y, 0.0)
        z = jnp.maximum(acc, 0.0)
        rows = i * BN + lax.broadcasted_iota(jnp.int32, (BN, 1), 0)
        z = jnp.where(rows < N_NODES, z, 0.0)
        z_ref[...] = z

        @pl.when(i == 0)
        def _init():
            s_ref[...] = jnp.zeros_like(s_ref)
            sq_ref[...] = jnp.zeros_like(sq_ref)

        s_ref[...] += jnp.sum(z, axis=0, keepdims=True)
        sq_ref[...] += jnp.sum(z * z, axis=0, keepdims=True)

    full = lambda shape: pl.BlockSpec(shape, lambda i: (0,) * len(shape))
    in_specs = [
        pl.BlockSpec((BN, Din), lambda i: (i, 0)),
        pl.BlockSpec((1, BN, Din), lambda i: (0, i, 0)),
        pl.BlockSpec((1, BN, Din), lambda i: (1, i, 0)),
        pl.BlockSpec((1, BN, 16), lambda i: (0, i, 0)),
        pl.BlockSpec((1, BN, 16), lambda i: (1, i, 0)),
        full((MAX_DEG, Din, H)),
        full((MAX_DEG, H)),
    ]
    args = [x, aggp, aggp, degp, degp, W, b]
    if affine:
        in_specs += [full((1, Din))] * 4
        args += list(stats)
    out_specs = (
        pl.BlockSpec((BN, H), lambda i: (i, 0)),
        pl.BlockSpec((1, H), lambda i: (0, 0)),
        pl.BlockSpec((1, H), lambda i: (0, 0)),
    )
    out_shape = (
        jax.ShapeDtypeStruct((NR, H), jnp.float32),
        jax.ShapeDtypeStruct((1, H), jnp.float32),
        jax.ShapeDtypeStruct((1, H), jnp.float32),
    )
    return pl.pallas_call(
        body, grid=(GRID,), in_specs=in_specs, out_specs=out_specs,
        out_shape=out_shape)(*args)


def _tc_dense(z2, s2, sq2, gamma2, beta2, Wd, bd):
    Din = z2.shape[1]
    H = Wd.shape[1]

    def body(z_ref, s_in, sq_in, ga, be, w_ref, b_ref,
             z_out, s_ref, sq_ref):
        i = pl.program_id(0)
        mean = s_in[...] * (1.0 / N_NODES)
        var = sq_in[...] * (1.0 / N_NODES) - mean * mean
        a = ga[...] * lax.rsqrt(var + EPS)
        cv = be[...] - mean * a
        h2 = a * z_ref[...] + cv
        h3 = jnp.dot(h2, w_ref[...], preferred_element_type=jnp.float32)
        h3 = h3 + b_ref[...]
        z_out[...] = h3
        rows = i * BN + lax.broadcasted_iota(jnp.int32, (BN, 1), 0)
        h3m = jnp.where(rows < N_NODES, h3, 0.0)

        @pl.when(i == 0)
        def _init():
            s_ref[...] = jnp.zeros_like(s_ref)
            sq_ref[...] = jnp.zeros_like(sq_ref)

        s_ref[...] += jnp.sum(h3m, axis=0, keepdims=True)
        sq_ref[...] += jnp.sum(h3m * h3m, axis=0, keepdims=True)

    full = lambda shape: pl.BlockSpec(shape, lambda i: (0,) * len(shape))
    return pl.pallas_call(
        body, grid=(GRID,),
        in_specs=[pl.BlockSpec((BN, Din), lambda i: (i, 0)),
                  full((1, Din)), full((1, Din)), full((1, Din)), full((1, Din)),
                  full((Din, H)), full((1, H))],
        out_specs=(pl.BlockSpec((BN, H), lambda i: (i, 0)),
                   pl.BlockSpec((1, H), lambda i: (0, 0)),
                   pl.BlockSpec((1, H), lambda i: (0, 0))),
        out_shape=(jax.ShapeDtypeStruct((NR, H), jnp.float32),
                   jax.ShapeDtypeStruct((1, H), jnp.float32),
                   jax.ShapeDtypeStruct((1, H), jnp.float32)),
    )(z2, s2, sq2, gamma2, beta2, Wd, bd)


def _tc_head(gsum, gmax, gmin, cnt, s3, sq3, gammad, betad, Wp, bp):
    D = gsum.shape[1]

    def body(gs_ref, gx_ref, gn_ref, c_ref, s_in, sq_in, ga, be,
             wp_ref, bp_ref, o_ref):
        mean = s_in[...] * (1.0 / N_NODES)
        var = sq_in[...] * (1.0 / N_NODES) - mean * mean
        a = ga[...] * lax.rsqrt(var + EPS)
        cv = be[...] - mean * a
        gs = a * gs_ref[...] + c_ref[:, :1] * cv
        gx = jnp.where(a > 0, a * gx_ref[...], a * gn_ref[...]) + cv
        t1 = jnp.tanh(gs)
        t2 = jnp.tanh(gx)
        o = jnp.dot(t1, wp_ref[0:D], preferred_element_type=jnp.float32)
        o = o + jnp.dot(t2, wp_ref[D:2 * D], preferred_element_type=jnp.float32)
        o_ref[...] = o + bp_ref[...]

    full = lambda a: pl.BlockSpec(a.shape, lambda: (0,) * a.ndim)
    args = (gsum, gmax, gmin, cnt, s3, sq3, gammad, betad, Wp, bp)
    return pl.pallas_call(
        body,
        in_specs=[full(a) for a in args],
        out_specs=pl.BlockSpec((N_GRAPHS, 1), lambda: (0, 0)),
        out_shape=jax.ShapeDtypeStruct((N_GRAPHS, 1), jnp.float32),
    )(*args)


def kernel(feats, edge_index, node_graph_ids, W1, b1, gamma1, beta1,
           W2, b2, gamma2, beta2, Wd, bd, gammad, betad, Wp, bp):
    feats_p = jnp.pad(feats, ((0, NR - N_NODES), (0, 0)))
    epad = NWORK * NCH * CHUNK - E_TOTAL
    src3 = jnp.pad(edge_index[0], (0, epad)).reshape(NWORK, NCH, CHUNK)
    dst3 = jnp.pad(edge_index[1], (0, epad),
                   constant_values=TRASH).reshape(NWORK, NCH, CHUNK)
    gids_p = jnp.pad(node_graph_ids, (0, NR - N_NODES),
                     constant_values=N_GRAPHS)

    agg1 = _sc_agg(feats_p, src3, dst3)
    degp = _sc_deg(dst3)
    z1, s1, sq1 = _tc_layer(feats_p, agg1, degp, W1, b1, None)

    agg2 = _sc_agg(z1, src3, dst3)
    z2, s2, sq2 = _tc_layer(
        z1, agg2, degp, W2, b2,
        (s1, sq1, gamma1.reshape(1, -1), beta1.reshape(1, -1)))

    z3, s3, sq3 = _tc_dense(z2, s2, sq2, gamma2.reshape(1, -1),
                            beta2.reshape(1, -1), Wd, bd.reshape(1, -1))

    gsum, gmax, gmin, cnt = _sc_readout(z3, gids_p)
    return _tc_head(gsum, gmax, gmin, cnt, s3, sq3, gammad.reshape(1, -1),
                    betad.reshape(1, -1), Wp, bp.reshape(1, -1))


# speedup vs baseline: 5.8390x; 1.3657x over previous
import functools

import jax
import jax.numpy as jnp
from jax import lax
from jax.experimental import pallas as pl
from jax.experimental.pallas import tpu as pltpu
from jax.experimental.pallas import tpu_sc as plsc

N_NODES = 10000
NR = 10240
N_GRAPHS = 64
MAX_DEG = 10
EPS = 1e-5
E_TOTAL = 320000
NWORK = 32
CHUNK = 128
NCH = 79
TRASH = N_NODES
BN = 1024
GRID = NR // BN
RSUB = NR // 16


def _make_sc_agg(D):
    mesh = plsc.VectorSubcoreMesh(core_axis_name="c", subcore_axis_name="s")
    scratch = [
        pltpu.VMEM((CHUNK,), jnp.int32),
        pltpu.VMEM((CHUNK,), jnp.int32),
        pltpu.VMEM((CHUNK,), jnp.int32),
        pltpu.VMEM((CHUNK,), jnp.int32),
        pltpu.VMEM((CHUNK, D), jnp.float32),
        pltpu.VMEM((CHUNK, D), jnp.float32),
        pltpu.VMEM_SHARED((NR, D), jnp.float32),
        pltpu.SemaphoreType.DMA,
        pltpu.SemaphoreType.DMA,
        pltpu.SemaphoreType.DMA,
    ]

    def body(table, src3, dst3, zrows, out_acc,
             sidxa, didxa, sidxb, didxb, rowsa, rowsb, acc,
             gsema, gsemb, isem):
        c = lax.axis_index("c")
        s = lax.axis_index("s")
        w = s * 2 + c
        sub = pl.ds(s * RSUB, RSUB)
        pltpu.sync_copy(zrows.at[sub], acc.at[sub])
        plsc.subcore_barrier()

        def fetch(j, sidx, didx, gsem, rows):
            pltpu.async_copy(src3.at[w, j], sidx, isem)
            pltpu.async_copy(dst3.at[w, j], didx, isem)
            pltpu.make_async_copy(src3.at[w, j], sidx, isem).wait()
            pltpu.make_async_copy(dst3.at[w, j], didx, isem).wait()
            pltpu.async_copy(table.at[sidx], rows, gsem)

        def scat(gsem, rows, didx):
            pltpu.make_async_copy(table.at[sidxa], rows, gsem).wait()
            pltpu.sync_copy(rows, acc.at[didx], add=True)

        fetch(0, sidxa, didxa, gsema, rowsa)

        @pl.loop(0, (NCH - 1) // 2)
        def step(t):
            fetch(2 * t + 1, sidxb, didxb, gsemb, rowsb)
            scat(gsema, rowsa, didxa)
            fetch(2 * t + 2, sidxa, didxa, gsema, rowsa)
            scat(gsemb, rowsb, didxb)

        scat(gsema, rowsa, didxa)
        plsc.subcore_barrier()
        pltpu.sync_copy(acc.at[sub], out_acc.at[c, sub])

    params = (pltpu.CompilerParams(use_tc_tiling_on_sc=False)
              if D != 128 else None)
    return pl.kernel(body, out_type=jax.ShapeDtypeStruct((2, NR, D),
                                                         jnp.float32),
                     mesh=mesh, scratch_types=scratch,
                     compiler_params=params)


def _sc_agg(table, src3, dst3):
    D = table.shape[1]
    k = _make_sc_agg(D)
    return k(table, src3, dst3, jnp.zeros((NR, D), jnp.float32))


def _make_sc_deg():
    mesh = plsc.VectorSubcoreMesh(core_axis_name="c", subcore_axis_name="s")
    scratch = [
        pltpu.VMEM((CHUNK,), jnp.int32),
        pltpu.VMEM((CHUNK, 16), jnp.float32),
        pltpu.VMEM_SHARED((NR, 16), jnp.float32),
        pltpu.VMEM((CHUNK, 16), jnp.float32),
        pltpu.VMEM((CHUNK,), jnp.int32),
        pltpu.VMEM((RSUB // 8, 128), jnp.float32),
        pltpu.SemaphoreType.DMA,
    ]

    def body(dst3, out_deg, didx, ones_v, dacc, zbuf, iv, pbuf, sem):
        c = lax.axis_index("c")
        s = lax.axis_index("s")
        w = s * 2 + c
        vone = jnp.ones((16,), jnp.float32)
        vnil = jnp.zeros((16,), jnp.float32)
        iota = lax.iota(jnp.int32, 16)
        for j in range(CHUNK):
            ones_v[j] = vone
            zbuf[j] = vnil
        for t in range(RSUB // CHUNK):
            for q in range(CHUNK // 16):
                iv[pl.ds(q * 16, 16)] = (s * RSUB + t * CHUNK + q * 16) + iota
            pltpu.sync_copy(zbuf, dacc.at[iv])
        plsc.subcore_barrier()

        def step(j, carry):
            pltpu.sync_copy(dst3.at[w, j], didx)
            pltpu.sync_copy(ones_v, dacc.at[didx], add=True)
            return carry

        lax.fori_loop(0, NCH, step, 0)
        plsc.subcore_barrier()
        for t in range(RSUB // CHUNK):
            for q in range(CHUNK // 16):
                iv[pl.ds(q * 16, 16)] = (s * RSUB + t * CHUNK + q * 16) + iota
            pltpu.async_copy(dacc.at[iv], zbuf, sem).wait()
            for j in range(CHUNK):
                jj = t * CHUNK + j
                pbuf[jj // 8, pl.ds((jj % 8) * 16, 16)] = zbuf[j, pl.ds(0, 16)]
        pltpu.sync_copy(pbuf,
                        out_deg.at[c, pl.ds(s * (RSUB // 8), RSUB // 8)])

    return pl.kernel(body,
                     out_type=jax.ShapeDtypeStruct((2, NR // 8, 128),
                                                   jnp.float32),
                     mesh=mesh, scratch_types=scratch,
                     compiler_params=pltpu.CompilerParams(
                         use_tc_tiling_on_sc=False))


def _sc_deg(dst3):
    return _make_sc_deg()(dst3).reshape(2, NR, 16)


RCH = 8


def _sc_readout(z3, gids_pad):
    D = z3.shape[1]
    nvec = D // 16
    mesh = plsc.VectorSubcoreMesh(core_axis_name="c", subcore_axis_name="s")
    outs = (
        jax.ShapeDtypeStruct((N_GRAPHS, D), jnp.float32),
        jax.ShapeDtypeStruct((N_GRAPHS, D), jnp.float32),
        jax.ShapeDtypeStruct((N_GRAPHS, D), jnp.float32),
        jax.ShapeDtypeStruct((N_GRAPHS, 16), jnp.float32),
    )
    scratch = [
        pltpu.VMEM((NR,), jnp.int32),
        pltpu.VMEM((RCH, D), jnp.float32),
        pltpu.VMEM((D,), jnp.float32),
        pltpu.VMEM((D,), jnp.float32),
        pltpu.VMEM((D,), jnp.float32),
        pltpu.VMEM((16,), jnp.float32),
    ]

    def body(z3h, gidh, osum, omax, omin, ocnt,
             gidv, buf, sacc, xacc, nacc, cntv):
        c = lax.axis_index("c")
        s = lax.axis_index("s")
        w = s * 2 + c
        g0 = w * 2
        pltpu.sync_copy(gidh, gidv)

        one = jnp.ones((16,), jnp.int32)
        nil = jnp.zeros((16,), jnp.int32)

        def cstep(i, carry):
            c0, c1, c2 = carry
            v = gidv[pl.ds(i * 16, 16)]
            c0 = c0 + jnp.where(v < g0, one, nil)
            c1 = c1 + jnp.where(v < (g0 + 1), one, nil)
            c2 = c2 + jnp.where(v < (g0 + 2), one, nil)
            return c0, c1, c2

        c0, c1, c2 = lax.fori_loop(0, NR // 16, cstep, (nil, nil, nil))
        b0 = jnp.int32(0)
        b1 = jnp.int32(0)
        b2 = jnp.int32(0)
        for j in range(16):
            b0 = b0 + c0[j]
            b1 = b1 + c1[j]
            b2 = b2 + c2[j]

        for k in range(2):
            g = g0 + k
            lo = b0 if k == 0 else b1
            hi = b1 if k == 0 else b2
            for t in range(nvec):
                sl = pl.ds(t * 16, 16)
                sacc[sl] = jnp.zeros((16,), jnp.float32)
                xacc[sl] = jnp.full((16,), -3e38, jnp.float32)
                nacc[sl] = jnp.full((16,), 3e38, jnp.float32)

            @pl.loop((lo // RCH) * RCH, hi, step=RCH)
            def rstep(r):
                rr = pl.multiple_of(r, RCH)
                pltpu.sync_copy(z3h.at[pl.ds(rr, RCH)], buf)
                for i in range(RCH):
                    valid = jnp.logical_and(r + i >= lo, r + i < hi)
                    for t in range(nvec):
                        sl = pl.ds(t * 16, 16)
                        row = buf[i, sl]
                        sacc[sl] = sacc[sl] + jnp.where(valid, row, 0.0)
                        xacc[sl] = jnp.maximum(xacc[sl],
                                               jnp.where(valid, row, -3e38))
                        nacc[sl] = jnp.minimum(nacc[sl],
                                               jnp.where(valid, row, 3e38))
            cntv[...] = jnp.full((16,), 1.0, jnp.float32) * (hi - lo).astype(jnp.float32)
            pltpu.sync_copy(sacc, osum.at[g])
            pltpu.sync_copy(xacc, omax.at[g])
            pltpu.sync_copy(nacc, omin.at[g])
            pltpu.sync_copy(cntv, ocnt.at[g])

    k = pl.kernel(body, out_type=outs, mesh=mesh, scratch_types=scratch)
    return k(z3, gids_pad)


def _tc_layer(x, aggp, degp, W, b, stats):
    Din = x.shape[1]
    H = W.shape[2]
    affine = stats is not None

    def body(*refs):
        if affine:
            (x_ref, p0, p1, d0, d1, w_ref, b_ref,
             s_in, sq_in, ga, be, z_ref, s_ref, sq_ref) = refs
        else:
            (x_ref, p0, p1, d0, d1, w_ref, b_ref,
             z_ref, s_ref, sq_ref) = refs
        i = pl.program_id(0)
        msg = x_ref[...] + p0[0] + p1[0]
        deg = d0[0][:, :1] + d1[0][:, :1]
        if affine:
            mean = s_in[...] * (1.0 / N_NODES)
            var = sq_in[...] * (1.0 / N_NODES) - mean * mean
            a = ga[...] * lax.rsqrt(var + EPS)
            cv = be[...] - mean * a
            msg = a * msg + (1.0 + deg) * cv
        didx = jnp.clip(deg, 1.0, float(MAX_DEG)) - 1.0
        acc = jnp.zeros((BN, H), jnp.float32)
        for d in range(MAX_DEG):
            y = jnp.dot(msg, w_ref[d], preferred_element_type=jnp.float32)
            y = y + b_ref[d]
            acc = acc + jnp.where(didx == float(d), y, 0.0)
        z = jnp.maximum(acc, 0.0)
        rows = i * BN + lax.broadcasted_iota(jnp.int32, (BN, 1), 0)
        z = jnp.where(rows < N_NODES, z, 0.0)
        z_ref[...] = z

        @pl.when(i == 0)
        def _init():
            s_ref[...] = jnp.zeros_like(s_ref)
            sq_ref[...] = jnp.zeros_like(sq_ref)

        s_ref[...] += jnp.sum(z, axis=0, keepdims=True)
        sq_ref[...] += jnp.sum(z * z, axis=0, keepdims=True)

    full = lambda shape: pl.BlockSpec(shape, lambda i: (0,) * len(shape))
    in_specs = [
        pl.BlockSpec((BN, Din), lambda i: (i, 0)),
        pl.BlockSpec((1, BN, Din), lambda i: (0, i, 0)),
        pl.BlockSpec((1, BN, Din), lambda i: (1, i, 0)),
        pl.BlockSpec((1, BN, 16), lambda i: (0, i, 0)),
        pl.BlockSpec((1, BN, 16), lambda i: (1, i, 0)),
        full((MAX_DEG, Din, H)),
        full((MAX_DEG, H)),
    ]
    args = [x, aggp, aggp, degp, degp, W, b]
    if affine:
        in_specs += [full((1, Din))] * 4
        args += list(stats)
    out_specs = (
        pl.BlockSpec((BN, H), lambda i: (i, 0)),
        pl.BlockSpec((1, H), lambda i: (0, 0)),
        pl.BlockSpec((1, H), lambda i: (0, 0)),
    )
    out_shape = (
        jax.ShapeDtypeStruct((NR, H), jnp.float32),
        jax.ShapeDtypeStruct((1, H), jnp.float32),
        jax.ShapeDtypeStruct((1, H), jnp.float32),
    )
    return pl.pallas_call(
        body, grid=(GRID,), in_specs=in_specs, out_specs=out_specs,
        out_shape=out_shape)(*args)


def _tc_dense(z2, s2, sq2, gamma2, beta2, Wd, bd):
    Din = z2.shape[1]
    H = Wd.shape[1]

    def body(z_ref, s_in, sq_in, ga, be, w_ref, b_ref,
             z_out, s_ref, sq_ref):
        i = pl.program_id(0)
        mean = s_in[...] * (1.0 / N_NODES)
        var = sq_in[...] * (1.0 / N_NODES) - mean * mean
        a = ga[...] * lax.rsqrt(var + EPS)
        cv = be[...] - mean * a
        h2 = a * z_ref[...] + cv
        h3 = jnp.dot(h2, w_ref[...], preferred_element_type=jnp.float32)
        h3 = h3 + b_ref[...]
        z_out[...] = h3
        rows = i * BN + lax.broadcasted_iota(jnp.int32, (BN, 1), 0)
        h3m = jnp.where(rows < N_NODES, h3, 0.0)

        @pl.when(i == 0)
        def _init():
            s_ref[...] = jnp.zeros_like(s_ref)
            sq_ref[...] = jnp.zeros_like(sq_ref)

        s_ref[...] += jnp.sum(h3m, axis=0, keepdims=True)
        sq_ref[...] += jnp.sum(h3m * h3m, axis=0, keepdims=True)

    full = lambda shape: pl.BlockSpec(shape, lambda i: (0,) * len(shape))
    return pl.pallas_call(
        body, grid=(GRID,),
        in_specs=[pl.BlockSpec((BN, Din), lambda i: (i, 0)),
                  full((1, Din)), full((1, Din)), full((1, Din)), full((1, Din)),
                  full((Din, H)), full((1, H))],
        out_specs=(pl.BlockSpec((BN, H), lambda i: (i, 0)),
                   pl.BlockSpec((1, H), lambda i: (0, 0)),
                   pl.BlockSpec((1, H), lambda i: (0, 0))),
        out_shape=(jax.ShapeDtypeStruct((NR, H), jnp.float32),
                   jax.ShapeDtypeStruct((1, H), jnp.float32),
                   jax.ShapeDtypeStruct((1, H), jnp.float32)),
    )(z2, s2, sq2, gamma2, beta2, Wd, bd)


def _tc_head(gsum, gmax, gmin, cnt, s3, sq3, gammad, betad, Wp, bp):
    D = gsum.shape[1]

    def body(gs_ref, gx_ref, gn_ref, c_ref, s_in, sq_in, ga, be,
             wp_ref, bp_ref, o_ref):
        mean = s_in[...] * (1.0 / N_NODES)
        var = sq_in[...] * (1.0 / N_NODES) - mean * mean
        a = ga[...] * lax.rsqrt(var + EPS)
        cv = be[...] - mean * a
        gs = a * gs_ref[...] + c_ref[:, :1] * cv
        gx = jnp.where(a > 0, a * gx_ref[...], a * gn_ref[...]) + cv
        t1 = jnp.tanh(gs)
        t2 = jnp.tanh(gx)
        o = jnp.dot(t1, wp_ref[0:D], preferred_element_type=jnp.float32)
        o = o + jnp.dot(t2, wp_ref[D:2 * D], preferred_element_type=jnp.float32)
        o_ref[...] = o + bp_ref[...]

    full = lambda a: pl.BlockSpec(a.shape, lambda: (0,) * a.ndim)
    args = (gsum, gmax, gmin, cnt, s3, sq3, gammad, betad, Wp, bp)
    return pl.pallas_call(
        body,
        in_specs=[full(a) for a in args],
        out_specs=pl.BlockSpec((N_GRAPHS, 1), lambda: (0, 0)),
        out_shape=jax.ShapeDtypeStruct((N_GRAPHS, 1), jnp.float32),
    )(*args)


def kernel(feats, edge_index, node_graph_ids, W1, b1, gamma1, beta1,
           W2, b2, gamma2, beta2, Wd, bd, gammad, betad, Wp, bp):
    feats_p = jnp.pad(feats, ((0, NR - N_NODES), (0, 0)))
    epad = NWORK * NCH * CHUNK - E_TOTAL
    src3 = jnp.pad(edge_index[0], (0, epad)).reshape(NWORK, NCH, CHUNK)
    pad_dst = (jnp.arange(epad, dtype=jnp.int32) % (NR - N_NODES)) + N_NODES
    dst3 = jnp.concatenate([edge_index[1], pad_dst]).reshape(
        NWORK, NCH, CHUNK)
    gids_p = jnp.pad(node_graph_ids, (0, NR - N_NODES),
                     constant_values=N_GRAPHS)

    agg1 = _sc_agg(feats_p, src3, dst3)
    degp = _sc_deg(dst3)
    z1, s1, sq1 = _tc_layer(feats_p, agg1, degp, W1, b1, None)

    agg2 = _sc_agg(z1, src3, dst3)
    z2, s2, sq2 = _tc_layer(
        z1, agg2, degp, W2, b2,
        (s1, sq1, gamma1.reshape(1, -1), beta1.reshape(1, -1)))

    z3, s3, sq3 = _tc_dense(z2, s2, sq2, gamma2.reshape(1, -1),
                            beta2.reshape(1, -1), Wd, bd.reshape(1, -1))

    gsum, gmax, gmin, cnt = _sc_readout(z3, gids_p)
    return _tc_head(gsum, gmax, gmin, cnt, s3, sq3, gammad.reshape(1, -1),
                    betad.reshape(1, -1), Wp, bp.reshape(1, -1))


# speedup vs baseline: 6.0477x; 1.0357x over previous
import functools

import jax
import jax.numpy as jnp
from jax import lax
from jax.experimental import pallas as pl
from jax.experimental.pallas import tpu as pltpu
from jax.experimental.pallas import tpu_sc as plsc

N_NODES = 10000
NR = 10240
N_GRAPHS = 64
MAX_DEG = 10
EPS = 1e-5
E_TOTAL = 320000
NWORK = 32
CHUNK = 128
NCH = 79
NCH0 = 105
NCH1 = 53
TRASH = N_NODES
BN = 1024
GRID = NR // BN
RSUB = NR // 16


def _make_sc_agg(D):
    mesh = plsc.VectorSubcoreMesh(core_axis_name="c", subcore_axis_name="s")
    scratch = [
        pltpu.VMEM((CHUNK,), jnp.int32),
        pltpu.VMEM((CHUNK,), jnp.int32),
        pltpu.VMEM((CHUNK,), jnp.int32),
        pltpu.VMEM((CHUNK,), jnp.int32),
        pltpu.VMEM((CHUNK, D), jnp.float32),
        pltpu.VMEM((CHUNK, D), jnp.float32),
        pltpu.VMEM_SHARED((NR, D), jnp.float32),
        pltpu.SemaphoreType.DMA,
        pltpu.SemaphoreType.DMA,
        pltpu.SemaphoreType.DMA,
    ]

    def body(table, src2, dst2, zrows, out_acc,
             sidxa, didxa, sidxb, didxb, rowsa, rowsb, acc,
             gsema, gsemb, isem):
        c = lax.axis_index("c")
        s = lax.axis_index("s")
        sub = pl.ds(s * RSUB, RSUB)
        pltpu.sync_copy(zrows.at[sub], acc.at[sub])
        plsc.subcore_barrier()

        base = jnp.where(c == 0, s * NCH0, 16 * NCH0 + s * NCH1)
        cnt = jnp.where(c == 0, NCH0, NCH1)

        def fetch(j, sidx, didx, gsem, rows):
            pltpu.async_copy(src2.at[j], sidx, isem)
            pltpu.async_copy(dst2.at[j], didx, isem)
            pltpu.make_async_copy(src2.at[j], sidx, isem).wait()
            pltpu.make_async_copy(dst2.at[j], didx, isem).wait()
            pltpu.async_copy(table.at[sidx], rows, gsem)

        def scat(gsem, rows, didx):
            pltpu.make_async_copy(table.at[sidxa], rows, gsem).wait()
            pltpu.sync_copy(rows, acc.at[didx], add=True)

        fetch(base, sidxa, didxa, gsema, rowsa)

        @pl.loop(0, (cnt - 1) // 2)
        def step(t):
            fetch(base + 2 * t + 1, sidxb, didxb, gsemb, rowsb)
            scat(gsema, rowsa, didxa)
            fetch(base + 2 * t + 2, sidxa, didxa, gsema, rowsa)
            scat(gsemb, rowsb, didxb)

        scat(gsema, rowsa, didxa)
        plsc.subcore_barrier()
        pltpu.sync_copy(acc.at[sub], out_acc.at[c, sub])

    params = (pltpu.CompilerParams(use_tc_tiling_on_sc=False)
              if D != 128 else None)
    return pl.kernel(body, out_type=jax.ShapeDtypeStruct((2, NR, D),
                                                         jnp.float32),
                     mesh=mesh, scratch_types=scratch,
                     compiler_params=params)


def _sc_agg(table, src3, dst3):
    D = table.shape[1]
    k = _make_sc_agg(D)
    return k(table, src3, dst3, jnp.zeros((NR, D), jnp.float32))


def _make_sc_deg():
    mesh = plsc.VectorSubcoreMesh(core_axis_name="c", subcore_axis_name="s")
    scratch = [
        pltpu.VMEM((CHUNK,), jnp.int32),
        pltpu.VMEM((CHUNK, 16), jnp.float32),
        pltpu.VMEM_SHARED((NR, 16), jnp.float32),
        pltpu.VMEM((CHUNK, 16), jnp.float32),
        pltpu.VMEM((CHUNK,), jnp.int32),
        pltpu.VMEM((RSUB // 8, 128), jnp.float32),
        pltpu.SemaphoreType.DMA,
    ]

    def body(dst3, out_deg, didx, ones_v, dacc, zbuf, iv, pbuf, sem):
        c = lax.axis_index("c")
        s = lax.axis_index("s")
        w = s * 2 + c
        vone = jnp.ones((16,), jnp.float32)
        vnil = jnp.zeros((16,), jnp.float32)
        iota = lax.iota(jnp.int32, 16)
        for j in range(CHUNK):
            ones_v[j] = vone
            zbuf[j] = vnil
        for t in range(RSUB // CHUNK):
            for q in range(CHUNK // 16):
                iv[pl.ds(q * 16, 16)] = (s * RSUB + t * CHUNK + q * 16) + iota
            pltpu.sync_copy(zbuf, dacc.at[iv])
        plsc.subcore_barrier()

        def step(j, carry):
            pltpu.sync_copy(dst3.at[w, j], didx)
            pltpu.sync_copy(ones_v, dacc.at[didx], add=True)
            return carry

        lax.fori_loop(0, NCH, step, 0)
        plsc.subcore_barrier()
        for t in range(RSUB // CHUNK):
            for q in range(CHUNK // 16):
                iv[pl.ds(q * 16, 16)] = (s * RSUB + t * CHUNK + q * 16) + iota
            pltpu.async_copy(dacc.at[iv], zbuf, sem).wait()
            for j in range(CHUNK):
                jj = t * CHUNK + j
                pbuf[jj // 8, pl.ds((jj % 8) * 16, 16)] = zbuf[j, pl.ds(0, 16)]
        pltpu.sync_copy(pbuf,
                        out_deg.at[c, pl.ds(s * (RSUB // 8), RSUB // 8)])

    return pl.kernel(body,
                     out_type=jax.ShapeDtypeStruct((2, NR // 8, 128),
                                                   jnp.float32),
                     mesh=mesh, scratch_types=scratch,
                     compiler_params=pltpu.CompilerParams(
                         use_tc_tiling_on_sc=False))


def _sc_deg(dst3):
    return _make_sc_deg()(dst3).reshape(2, NR, 16)


RCH = 8


def _sc_readout(z3, gids_pad):
    D = z3.shape[1]
    nvec = D // 16
    mesh = plsc.VectorSubcoreMesh(core_axis_name="c", subcore_axis_name="s")
    outs = (
        jax.ShapeDtypeStruct((N_GRAPHS, D), jnp.float32),
        jax.ShapeDtypeStruct((N_GRAPHS, D), jnp.float32),
        jax.ShapeDtypeStruct((N_GRAPHS, D), jnp.float32),
        jax.ShapeDtypeStruct((N_GRAPHS, 16), jnp.float32),
    )
    scratch = [
        pltpu.VMEM((NR,), jnp.int32),
        pltpu.VMEM((RCH, D), jnp.float32),
        pltpu.VMEM((D,), jnp.float32),
        pltpu.VMEM((D,), jnp.float32),
        pltpu.VMEM((D,), jnp.float32),
        pltpu.VMEM((16,), jnp.float32),
    ]

    def body(z3h, gidh, osum, omax, omin, ocnt,
             gidv, buf, sacc, xacc, nacc, cntv):
        c = lax.axis_index("c")
        s = lax.axis_index("s")
        w = s * 2 + c
        g0 = w * 2
        pltpu.sync_copy(gidh, gidv)

        one = jnp.ones((16,), jnp.int32)
        nil = jnp.zeros((16,), jnp.int32)

        def cstep(i, carry):
            c0, c1, c2 = carry
            v = gidv[pl.ds(i * 16, 16)]
            c0 = c0 + jnp.where(v < g0, one, nil)
            c1 = c1 + jnp.where(v < (g0 + 1), one, nil)
            c2 = c2 + jnp.where(v < (g0 + 2), one, nil)
            return c0, c1, c2

        c0, c1, c2 = lax.fori_loop(0, NR // 16, cstep, (nil, nil, nil))
        b0 = jnp.int32(0)
        b1 = jnp.int32(0)
        b2 = jnp.int32(0)
        for j in range(16):
            b0 = b0 + c0[j]
            b1 = b1 + c1[j]
            b2 = b2 + c2[j]

        for k in range(2):
            g = g0 + k
            lo = b0 if k == 0 else b1
            hi = b1 if k == 0 else b2
            for t in range(nvec):
                sl = pl.ds(t * 16, 16)
                sacc[sl] = jnp.zeros((16,), jnp.float32)
                xacc[sl] = jnp.full((16,), -3e38, jnp.float32)
                nacc[sl] = jnp.full((16,), 3e38, jnp.float32)

            @pl.loop((lo // RCH) * RCH, hi, step=RCH)
            def rstep(r):
                rr = pl.multiple_of(r, RCH)
                pltpu.sync_copy(z3h.at[pl.ds(rr, RCH)], buf)
                for i in range(RCH):
                    valid = jnp.logical_and(r + i >= lo, r + i < hi)
                    for t in range(nvec):
                        sl = pl.ds(t * 16, 16)
                        row = buf[i, sl]
                        sacc[sl] = sacc[sl] + jnp.where(valid, row, 0.0)
                        xacc[sl] = jnp.maximum(xacc[sl],
                                               jnp.where(valid, row, -3e38))
                        nacc[sl] = jnp.minimum(nacc[sl],
                                               jnp.where(valid, row, 3e38))
            cntv[...] = jnp.full((16,), 1.0, jnp.float32) * (hi - lo).astype(jnp.float32)
            pltpu.sync_copy(sacc, osum.at[g])
            pltpu.sync_copy(xacc, omax.at[g])
            pltpu.sync_copy(nacc, omin.at[g])
            pltpu.sync_copy(cntv, ocnt.at[g])

    k = pl.kernel(body, out_type=outs, mesh=mesh, scratch_types=scratch)
    return k(z3, gids_pad)


def _tc_layer(x, aggp, degp, W, b, stats):
    Din = x.shape[1]
    H = W.shape[2]
    affine = stats is not None

    def body(*refs):
        if affine:
            (x_ref, p0, p1, d0, d1, w_ref, b_ref,
             s_in, sq_in, ga, be, z_ref, s_ref, sq_ref) = refs
        else:
            (x_ref, p0, p1, d0, d1, w_ref, b_ref,
             z_ref, s_ref, sq_ref) = refs
        i = pl.program_id(0)
        msg = x_ref[...] + p0[0] + p1[0]
        deg = d0[0][:, :1] + d1[0][:, :1]
        if affine:
            mean = s_in[...] * (1.0 / N_NODES)
            var = sq_in[...] * (1.0 / N_NODES) - mean * mean
            a = ga[...] * lax.rsqrt(var + EPS)
            cv = be[...] - mean * a
            msg = a * msg + (1.0 + deg) * cv
        didx = jnp.clip(deg, 1.0, float(MAX_DEG)) - 1.0
        acc = jnp.zeros((BN, H), jnp.float32)
        for d in range(MAX_DEG):
            y = jnp.dot(msg, w_ref[d], preferred_element_type=jnp.float32)
            y = y + b_ref[d]
            acc = acc + jnp.where(didx == float(d), y, 0.0)
        z = jnp.maximum(acc, 0.0)
        rows = i * BN + lax.broadcasted_iota(jnp.int32, (BN, 1), 0)
        z = jnp.where(rows < N_NODES, z, 0.0)
        z_ref[...] = z

        @pl.when(i == 0)
        def _init():
            s_ref[...] = jnp.zeros_like(s_ref)
            sq_ref[...] = jnp.zeros_like(sq_ref)

        s_ref[...] += jnp.sum(z, axis=0, keepdims=True)
        sq_ref[...] += jnp.sum(z * z, axis=0, keepdims=True)

    full = lambda shape: pl.BlockSpec(shape, lambda i: (0,) * len(shape))
    in_specs = [
        pl.BlockSpec((BN, Din), lambda i: (i, 0)),
        pl.BlockSpec((1, BN, Din), lambda i: (0, i, 0)),
        pl.BlockSpec((1, BN, Din), lambda i: (1, i, 0)),
        pl.BlockSpec((1, BN, 16), lambda i: (0, i, 0)),
        pl.BlockSpec((1, BN, 16), lambda i: (1, i, 0)),
        full((MAX_DEG, Din, H)),
        full((MAX_DEG, H)),
    ]
    args = [x, aggp, aggp, degp, degp, W, b]
    if affine:
        in_specs += [full((1, Din))] * 4
        args += list(stats)
    out_specs = (
        pl.BlockSpec((BN, H), lambda i: (i, 0)),
        pl.BlockSpec((1, H), lambda i: (0, 0)),
        pl.BlockSpec((1, H), lambda i: (0, 0)),
    )
    out_shape = (
        jax.ShapeDtypeStruct((NR, H), jnp.float32),
        jax.ShapeDtypeStruct((1, H), jnp.float32),
        jax.ShapeDtypeStruct((1, H), jnp.float32),
    )
    return pl.pallas_call(
        body, grid=(GRID,), in_specs=in_specs, out_specs=out_specs,
        out_shape=out_shape)(*args)


def _tc_dense(z2, s2, sq2, gamma2, beta2, Wd, bd):
    Din = z2.shape[1]
    H = Wd.shape[1]

    def body(z_ref, s_in, sq_in, ga, be, w_ref, b_ref,
             z_out, s_ref, sq_ref):
        i = pl.program_id(0)
        mean = s_in[...] * (1.0 / N_NODES)
        var = sq_in[...] * (1.0 / N_NODES) - mean * mean
        a = ga[...] * lax.rsqrt(var + EPS)
        cv = be[...] - mean * a
        h2 = a * z_ref[...] + cv
        h3 = jnp.dot(h2, w_ref[...], preferred_element_type=jnp.float32)
        h3 = h3 + b_ref[...]
        z_out[...] = h3
        rows = i * BN + lax.broadcasted_iota(jnp.int32, (BN, 1), 0)
        h3m = jnp.where(rows < N_NODES, h3, 0.0)

        @pl.when(i == 0)
        def _init():
            s_ref[...] = jnp.zeros_like(s_ref)
            sq_ref[...] = jnp.zeros_like(sq_ref)

        s_ref[...] += jnp.sum(h3m, axis=0, keepdims=True)
        sq_ref[...] += jnp.sum(h3m * h3m, axis=0, keepdims=True)

    full = lambda shape: pl.BlockSpec(shape, lambda i: (0,) * len(shape))
    return pl.pallas_call(
        body, grid=(GRID,),
        in_specs=[pl.BlockSpec((BN, Din), lambda i: (i, 0)),
                  full((1, Din)), full((1, Din)), full((1, Din)), full((1, Din)),
                  full((Din, H)), full((1, H))],
        out_specs=(pl.BlockSpec((BN, H), lambda i: (i, 0)),
                   pl.BlockSpec((1, H), lambda i: (0, 0)),
                   pl.BlockSpec((1, H), lambda i: (0, 0))),
        out_shape=(jax.ShapeDtypeStruct((NR, H), jnp.float32),
                   jax.ShapeDtypeStruct((1, H), jnp.float32),
                   jax.ShapeDtypeStruct((1, H), jnp.float32)),
    )(z2, s2, sq2, gamma2, beta2, Wd, bd)


def _tc_head(gsum, gmax, gmin, cnt, s3, sq3, gammad, betad, Wp, bp):
    D = gsum.shape[1]

    def body(gs_ref, gx_ref, gn_ref, c_ref, s_in, sq_in, ga, be,
             wp_ref, bp_ref, o_ref):
        mean = s_in[...] * (1.0 / N_NODES)
        var = sq_in[...] * (1.0 / N_NODES) - mean * mean
        a = ga[...] * lax.rsqrt(var + EPS)
        cv = be[...] - mean * a
        gs = a * gs_ref[...] + c_ref[:, :1] * cv
        gx = jnp.where(a > 0, a * gx_ref[...], a * gn_ref[...]) + cv
        t1 = jnp.tanh(gs)
        t2 = jnp.tanh(gx)
        o = jnp.dot(t1, wp_ref[0:D], preferred_element_type=jnp.float32)
        o = o + jnp.dot(t2, wp_ref[D:2 * D], preferred_element_type=jnp.float32)
        o_ref[...] = o + bp_ref[...]

    full = lambda a: pl.BlockSpec(a.shape, lambda: (0,) * a.ndim)
    args = (gsum, gmax, gmin, cnt, s3, sq3, gammad, betad, Wp, bp)
    return pl.pallas_call(
        body,
        in_specs=[full(a) for a in args],
        out_specs=pl.BlockSpec((N_GRAPHS, 1), lambda: (0, 0)),
        out_shape=jax.ShapeDtypeStruct((N_GRAPHS, 1), jnp.float32),
    )(*args)


def kernel(feats, edge_index, node_graph_ids, W1, b1, gamma1, beta1,
           W2, b2, gamma2, beta2, Wd, bd, gammad, betad, Wp, bp):
    feats_p = jnp.pad(feats, ((0, NR - N_NODES), (0, 0)))
    epad = NWORK * NCH * CHUNK - E_TOTAL
    srcp = jnp.pad(edge_index[0], (0, epad))
    pad_dst = (jnp.arange(epad, dtype=jnp.int32) % (NR - N_NODES)) + N_NODES
    dstp = jnp.concatenate([edge_index[1], pad_dst])
    src2 = srcp.reshape(NWORK * NCH, CHUNK)
    dst2 = dstp.reshape(NWORK * NCH, CHUNK)
    dst3 = dstp.reshape(NWORK, NCH, CHUNK)
    gids_p = jnp.pad(node_graph_ids, (0, NR - N_NODES),
                     constant_values=N_GRAPHS)

    agg1 = _sc_agg(feats_p, src2, dst2)
    degp = _sc_deg(dst3)
    z1, s1, sq1 = _tc_layer(feats_p, agg1, degp, W1, b1, None)

    agg2 = _sc_agg(z1, src2, dst2)
    z2, s2, sq2 = _tc_layer(
        z1, agg2, degp, W2, b2,
        (s1, sq1, gamma1.reshape(1, -1), beta1.reshape(1, -1)))

    z3, s3, sq3 = _tc_dense(z2, s2, sq2, gamma2.reshape(1, -1),
                            beta2.reshape(1, -1), Wd, bd.reshape(1, -1))

    gsum, gmax, gmin, cnt = _sc_readout(z3, gids_p)
    return _tc_head(gsum, gmax, gmin, cnt, s3, sq3, gammad.reshape(1, -1),
                    betad.reshape(1, -1), Wp, bp.reshape(1, -1))
